# Initial kernel scaffold; baseline (speedup 1.0000x reference)
import jax
import jax.numpy as jnp
from jax.experimental import pallas as pl


def kernel(x, edge_index, edge_attr, batch, W1, b1, W2, b2, W3, b3, Wl, bl):
    raise NotImplementedError("write your pallas kernel here")


# speedup vs baseline: 14.5200x; 14.5200x over previous
import functools

import jax
import jax.numpy as jnp
from jax import lax
from jax.experimental import pallas as pl
from jax.experimental.pallas import tpu as pltpu
from jax.experimental.pallas import tpu_sc as plsc

N = 100000
NP = 100096
E = 3200000
G = 512
HP = 8
NC = 2
NS = 16
NW = NC * NS
EPW = E // NW
C = 1000
NCH = EPW // C
RPT = NP // NS
NF = NP // 16
NROWS = N // 16
PPW = NP // NW

_mesh = plsc.VectorSubcoreMesh(core_axis_name="c", subcore_axis_name="s")
_sc_params = pltpu.CompilerParams(needs_layout_passes=False,
                                  use_tc_tiling_on_sc=False)


def _patterns():
    iot = lax.broadcasted_iota(jnp.int32, (16,), 0)
    return iot >> 3, iot & 7


def _zero_rows8(ref, nrows):
    rp, cp = _patterns()
    z = jnp.zeros((16,), jnp.float32)

    def st(i, _):
        plsc.store_scatter(ref, [rp + 2 * i, cp], z)
        return 0

    lax.fori_loop(0, nrows // 2, st, 0)


@functools.partial(
    pl.kernel,
    out_type=jax.ShapeDtypeStruct((2 * NP, HP), jnp.float32),
    mesh=_mesh,
    compiler_params=_sc_params,
    scratch_types=[
        pltpu.VMEM_SHARED((NP, HP), jnp.float32),
        pltpu.VMEM_SHARED((NP, HP), jnp.float32),
        pltpu.VMEM((C,), jnp.int32),
        pltpu.VMEM((C,), jnp.int32),
        pltpu.VMEM((C * HP,), jnp.float32),
        pltpu.VMEM((C, HP), jnp.float32),
        pltpu.SemaphoreType.DMA,
    ],
)
def _sc_aggr(h_hbm, src_hbm, dst_hbm, ea_hbm, out_hbm,
             h_sp, acc_sp, sidx, didx, ebuf, gbuf, gsem):
    c = lax.axis_index("c")
    s = lax.axis_index("s")
    wid = c * NS + s

    r0 = s * RPT
    pltpu.sync_copy(h_hbm.at[pl.ds(r0, RPT)], h_sp.at[pl.ds(r0, RPT)])
    _zero_rows8(gbuf, C)
    for k in range(RPT // C):
        pltpu.sync_copy(gbuf, acc_sp.at[pl.ds(r0 + k * C, C)])
    rem = RPT % C
    if rem:
        pltpu.sync_copy(gbuf.at[pl.ds(0, rem)],
                        acc_sp.at[pl.ds(r0 + (RPT // C) * C, rem)])
    plsc.subcore_barrier()

    ebase = wid * EPW
    rp, cp = _patterns()

    def chunk(i, _):
        off = ebase + i * C
        pltpu.sync_copy(src_hbm.at[pl.ds(off, C)], sidx)
        pltpu.sync_copy(dst_hbm.at[pl.ds(off, C)], didx)
        pltpu.sync_copy(ea_hbm.at[pl.ds(off * HP, C * HP)], ebuf)
        pltpu.async_copy(h_sp.at[sidx], gbuf, gsem).wait()

        def vstep(j, _):
            ridx = rp + 2 * j
            g = plsc.load_gather(gbuf, [ridx, cp])
            m = jnp.maximum(g + ebuf[pl.ds(j * 16, 16)], 0.0)
            plsc.store_scatter(gbuf, [ridx, cp], m)
            return 0

        lax.fori_loop(0, C * HP // 16, vstep, 0)
        pltpu.sync_copy(gbuf, acc_sp.at[didx], add=True)
        return 0

    lax.fori_loop(0, NCH, chunk, 0)
    plsc.subcore_barrier()

    pltpu.sync_copy(acc_sp.at[pl.ds(r0, RPT)],
                    out_hbm.at[pl.ds(c * NP + r0, RPT)])


@functools.partial(
    pl.kernel,
    out_type=jax.ShapeDtypeStruct((2 * G, HP), jnp.float32),
    mesh=_mesh,
    compiler_params=_sc_params,
    scratch_types=[
        pltpu.VMEM_SHARED((G, HP), jnp.float32),
        pltpu.VMEM((PPW,), jnp.int32),
        pltpu.VMEM((PPW, HP), jnp.float32),
        pltpu.VMEM((G, HP), jnp.float32),
    ],
)
def _sc_pool(h_hbm, b_hbm, out_hbm, acc_sp, bidx, rows, zbuf):
    c = lax.axis_index("c")
    s = lax.axis_index("s")
    wid = c * NS + s

    @pl.when(s == 0)
    def _():
        _zero_rows8(zbuf, G)
        pltpu.sync_copy(zbuf, acc_sp)

    plsc.subcore_barrier()
    r0 = wid * PPW
    pltpu.sync_copy(b_hbm.at[pl.ds(r0, PPW)], bidx)
    pltpu.sync_copy(h_hbm.at[pl.ds(r0, PPW)], rows)
    pltpu.sync_copy(rows, acc_sp.at[bidx], add=True)
    plsc.subcore_barrier()

    @pl.when(s == 0)
    def _():
        pltpu.sync_copy(acc_sp, out_hbm.at[pl.ds(c * G, G)])


def _tc_update_body(h_ref, p_ref, w_ref, b_ref, o_ref, *, relu, ones_col):
    t = h_ref[...] + p_ref[0] + p_ref[1]
    y = jnp.dot(t, w_ref[...], preferred_element_type=jnp.float32) + b_ref[...]
    if relu:
        y = jnp.maximum(y, 0.0)
    if ones_col:
        lane = lax.broadcasted_iota(jnp.int32, (NF, 128), 1)
        row = lax.broadcasted_iota(jnp.int32, (NF, 128), 0)
        y = jnp.where(lane % HP == 6, 1.0, y)
        y = jnp.where(row < NROWS, y, 0.0)
    o_ref[...] = y


def _tc_update(hf, part, bd, bt, relu, ones_col):
    return pl.pallas_call(
        functools.partial(_tc_update_body, relu=relu, ones_col=ones_col),
        out_shape=jax.ShapeDtypeStruct((NF, 128), jnp.float32),
    )(hf, part, bd, bt)


def _tc_final_body(p_ref, wl_ref, bl_ref, o_ref):
    t = p_ref[0] + p_ref[1]
    cnt = jnp.maximum(t[:, 6:7], 1.0)
    pooled = t / cnt
    o_ref[...] = jnp.sum(pooled * wl_ref[...], axis=1, keepdims=True) + bl_ref[...]


def kernel(x, edge_index, edge_attr, batch, W1, b1, W2, b2, W3, b3, Wl, bl):
    f32 = jnp.float32
    src = edge_index[0].astype(jnp.int32)
    dst = edge_index[1].astype(jnp.int32)

    x8 = jnp.pad(x.astype(f32), ((0, NP - N), (0, HP - x.shape[1])))
    ea8 = jnp.pad(edge_attr.astype(f32), ((0, 0), (0, HP - edge_attr.shape[1])))
    ea8 = ea8.reshape(-1)
    batchp = jnp.pad(batch.astype(jnp.int32), (0, NP - N))

    eye16 = jnp.eye(16, dtype=f32)

    def expand(Wk, bk):
        w8 = jnp.zeros((HP, HP), f32).at[:6, :6].set(Wk.T.astype(f32))
        bd = jnp.kron(eye16, w8)
        bt = jnp.tile(jnp.pad(bk.astype(f32), (0, HP - 6)), 16).reshape(1, 128)
        return bd, bt

    bd1, bt1 = expand(W1, b1)
    bd2, bt2 = expand(W2, b2)
    bd3, bt3 = expand(W3, b3)

    h = x8
    for bd, bt, relu, ones_col in ((bd1, bt1, True, False),
                                   (bd2, bt2, True, False),
                                   (bd3, bt3, False, True)):
        part = _sc_aggr(h, src, dst, ea8)
        part = part.reshape(2, NF, 128)
        hf = h.reshape(NF, 128)
        h = _tc_update(hf, part, bd, bt, relu, ones_col).reshape(NP, HP)

    pool = _sc_pool(h, batchp).reshape(2, G, HP)

    wl8 = jnp.pad(Wl[0].astype(f32), (0, HP - 6)).reshape(1, HP)
    blr = bl.astype(f32).reshape(1, 1)
    out = pl.pallas_call(
        _tc_final_body,
        out_shape=jax.ShapeDtypeStruct((G, 1), jnp.float32),
    )(pool, wl8, blr)
    return out
